# Initial kernel scaffold; baseline (speedup 1.0000x reference)
import jax
import jax.numpy as jnp
from jax.experimental import pallas as pl


def kernel(im_data, im_info, W_head, b_head, W_rpn, b_rpn, W_score, b_score, W_bbox, b_bbox):
    raise NotImplementedError("write your pallas kernel here")


# speedup vs baseline: 6.4357x; 6.4357x over previous
import functools

import jax
import jax.numpy as jnp
import numpy as np
from jax.experimental import pallas as pl
from jax.experimental.pallas import tpu as pltpu

IMG_H, IMG_W = 1024, 1024
STRIDE = 16
FEAT = 64
P = FEAT * FEAT
NA = 9
N = P * NA
PRE_NMS_TOPN = 6000
POST_NMS_TOPN = 300
NMS_THRESH = 0.7
ROWS = N // 128
MBLK = 512


def _mk(ws, hs, x_ctr, y_ctr):
    ws = ws[:, None]
    hs = hs[:, None]
    return np.hstack([x_ctr - 0.5 * (ws - 1.0), y_ctr - 0.5 * (hs - 1.0),
                      x_ctr + 0.5 * (ws - 1.0), y_ctr + 0.5 * (hs - 1.0)])


def _base_anchors(base_size=16, ratios=(0.5, 1.0, 2.0), scales=(8, 16, 32)):
    ratios = np.array(ratios, dtype=np.float64)
    scales = np.array(scales, dtype=np.float64)
    w = h = float(base_size)
    x_ctr = y_ctr = 0.5 * (w - 1.0)
    size = w * h
    ws = np.round(np.sqrt(size / ratios))
    hs = np.round(ws * ratios)
    ra = _mk(ws, hs, x_ctr, y_ctr)
    out = []
    for i in range(ra.shape[0]):
        a = ra[i]
        wi = a[2] - a[0] + 1.0
        hi = a[3] - a[1] + 1.0
        xc = a[0] + 0.5 * (wi - 1.0)
        yc = a[1] + 0.5 * (hi - 1.0)
        out.append(_mk(wi * scales, hi * scales, xc, yc))
    return np.vstack(out).astype(np.float32)


def _all_anchors():
    base = _base_anchors()
    shift = np.arange(FEAT) * STRIDE
    sx, sy = np.meshgrid(shift, shift)
    shifts = np.stack([sx.ravel(), sy.ravel(), sx.ravel(), sy.ravel()],
                      axis=1).astype(np.float32)
    aa = (base[None, :, :] + shifts[:, None, :]).reshape(-1, 4)
    return aa.reshape(P, NA, 4)


_ANCH = _all_anchors()


def _k1_body(a_ref, w_ref, b_ref, o_ref):
    acc = jnp.dot(a_ref[:], w_ref[:], preferred_element_type=jnp.float32)
    o_ref[:] = jnp.maximum(acc + b_ref[0:1, :], 0.0)


def _head_conv(patches, w, b):
    nblk = P // MBLK
    return pl.pallas_call(
        _k1_body,
        grid=(nblk,),
        in_specs=[
            pl.BlockSpec((MBLK, 768), lambda i: (i, 0)),
            pl.BlockSpec((768, 512), lambda i: (0, 0)),
            pl.BlockSpec((8, 512), lambda i: (0, 0)),
        ],
        out_specs=pl.BlockSpec((MBLK, 512), lambda i: (i, 0)),
        out_shape=jax.ShapeDtypeStruct((P, 512), jnp.float32),
    )(patches, w, b)


MBLK2 = 256


def _k2_body(x_ref, w_ref, b_ref, o_ref):
    acc = jnp.dot(x_ref[:], w_ref[:], preferred_element_type=jnp.float32)
    o_ref[:] = acc + b_ref[0:1, :]


def _rpn_conv(xcol, w2, b):
    nblk = P // MBLK2
    return pl.pallas_call(
        _k2_body,
        grid=(nblk,),
        in_specs=[
            pl.BlockSpec((MBLK2, 9 * 512), lambda i: (i, 0)),
            pl.BlockSpec((9 * 512, 512), lambda i: (0, 0)),
            pl.BlockSpec((8, 512), lambda i: (0, 0)),
        ],
        out_specs=pl.BlockSpec((MBLK2, 512), lambda i: (i, 0)),
        out_shape=jax.ShapeDtypeStruct((P, 512), jnp.float32),
    )(xcol, w2, b)


def _k3_body(x_ref, wc_ref, bc_ref, ax1_ref, ay1_ref, ax2_ref, ay2_ref,
             hw_ref, sc_ref, x1_ref, y1_ref, x2_ref, y2_ref):
    s = jnp.dot(x_ref[:], wc_ref[:], preferred_element_type=jnp.float32)
    s = s + bc_ref[0:1, :]
    sbg = s[:, 0:9]
    sfg = s[:, 9:18]
    dx = s[:, 18:27]
    dy = s[:, 27:36]
    dw = s[:, 36:45]
    dh = s[:, 45:54]
    sc_ref[:] = 1.0 / (1.0 + jnp.exp(sbg - sfg))

    ax1 = ax1_ref[:]
    ay1 = ay1_ref[:]
    ax2 = ax2_ref[:]
    ay2 = ay2_ref[:]
    wa = ax2 - ax1 + 1.0
    ha = ay2 - ay1 + 1.0
    cx = ax1 + 0.5 * wa
    cy = ay1 + 0.5 * ha
    pcx = dx * wa + cx
    pcy = dy * ha + cy
    pw = jnp.exp(dw) * wa
    ph = jnp.exp(dh) * ha
    imh = hw_ref[0, 0]
    imw = hw_ref[0, 1]
    x1_ref[:] = jnp.clip(pcx - 0.5 * pw, 0.0, imw - 1.0)
    y1_ref[:] = jnp.clip(pcy - 0.5 * ph, 0.0, imh - 1.0)
    x2_ref[:] = jnp.clip(pcx + 0.5 * pw, 0.0, imw - 1.0)
    y2_ref[:] = jnp.clip(pcy + 0.5 * ph, 0.0, imh - 1.0)


def _heads(x, wc, bc, ax1, ay1, ax2, ay2, hw):
    nblk = P // MBLK
    plane = jax.ShapeDtypeStruct((P, NA), jnp.float32)
    bspec = pl.BlockSpec((MBLK, NA), lambda i: (i, 0))
    return pl.pallas_call(
        _k3_body,
        grid=(nblk,),
        in_specs=[
            pl.BlockSpec((MBLK, 512), lambda i: (i, 0)),
            pl.BlockSpec((512, 64), lambda i: (0, 0)),
            pl.BlockSpec((8, 64), lambda i: (0, 0)),
            bspec, bspec, bspec, bspec,
            pl.BlockSpec((8, 128), lambda i: (0, 0)),
        ],
        out_specs=[bspec, bspec, bspec, bspec, bspec],
        out_shape=[plane, plane, plane, plane, plane],
    )(x, wc, bc, ax1, ay1, ax2, ay2, hw)


def _k4_body(s_ref, x1_ref, y1_ref, x2_ref, y2_ref, o_ref, ms_ref):
    s = s_ref[:]
    x1 = x1_ref[:]
    y1 = y1_ref[:]
    x2 = x2_ref[:]
    y2 = y2_ref[:]
    area = (x2 - x1 + 1.0) * (y2 - y1 + 1.0)

    si = jax.lax.bitcast_convert_type(s, jnp.int32)

    def bis(_, lohi):
        lo, hi = lohi
        mid = (lo + hi) // 2
        cnt = jnp.sum((si >= mid).astype(jnp.int32))
        big = cnt >= PRE_NMS_TOPN
        return jnp.where(big, mid, lo), jnp.where(big, hi, mid)

    lo0 = jnp.int32(0)
    hi0 = jnp.int32(0x40000000)
    lo, _ = jax.lax.fori_loop(0, 31, bis, (lo0, hi0))
    ms_ref[:] = jnp.where(si >= lo, s, -1.0)

    rows = jax.lax.broadcasted_iota(jnp.int32, (ROWS, 128), 0)
    lanes = jax.lax.broadcasted_iota(jnp.int32, (ROWS, 128), 1)
    flat = rows * 128 + lanes
    lane1 = jax.lax.broadcasted_iota(jnp.int32, (1, 128), 1)

    def pick(ref, r, c):
        row = ref[pl.ds(r, 1), :]
        return jnp.sum(jnp.where(lane1 == c, row, 0.0))

    def step(t, j0):
        masked = ms_ref[:]
        m = jnp.max(masked)
        cand = jnp.where(masked == m, flat, jnp.int32(1 << 30))
        jmin = jnp.min(cand)
        anyv = m > 0.0
        j = jnp.where(anyv, jmin, j0)
        j0 = jnp.where(t == 0, j, j0)
        r = j // 128
        c = j - r * 128
        x1j = pick(x1_ref, r, c)
        y1j = pick(y1_ref, r, c)
        x2j = pick(x2_ref, r, c)
        y2j = pick(y2_ref, r, c)
        sj = pick(s_ref, r, c)
        aj = (x2j - x1j + 1.0) * (y2j - y1j + 1.0)
        xx1 = jnp.maximum(x1, x1j)
        yy1 = jnp.maximum(y1, y1j)
        xx2 = jnp.minimum(x2, x2j)
        yy2 = jnp.minimum(y2, y2j)
        w = jnp.maximum(xx2 - xx1 + 1.0, 0.0)
        h = jnp.maximum(yy2 - yy1 + 1.0, 0.0)
        inter = w * h
        iou = inter / (aj + area - inter)
        ms_ref[:] = jnp.where(iou <= NMS_THRESH, masked, -1.0)
        rowv = (jnp.where(lane1 == 0, x1j, 0.0) +
                jnp.where(lane1 == 1, y1j, 0.0) +
                jnp.where(lane1 == 2, x2j, 0.0) +
                jnp.where(lane1 == 3, y2j, 0.0) +
                jnp.where(lane1 == 4, sj, 0.0))
        o_ref[pl.ds(t, 1), :] = rowv
        return j0

    jax.lax.fori_loop(0, POST_NMS_TOPN, step, jnp.int32(0))


def _nms(sf, x1f, y1f, x2f, y2f):
    fspec = pl.BlockSpec((ROWS, 128), lambda: (0, 0))
    return pl.pallas_call(
        _k4_body,
        grid=(),
        in_specs=[fspec, fspec, fspec, fspec, fspec],
        out_specs=pl.BlockSpec((POST_NMS_TOPN, 128), lambda: (0, 0)),
        out_shape=jax.ShapeDtypeStruct((POST_NMS_TOPN, 128), jnp.float32),
        scratch_shapes=[pltpu.VMEM((ROWS, 128), jnp.float32)],
    )(sf, x1f, y1f, x2f, y2f)


def kernel(im_data, im_info, W_head, b_head, W_rpn, b_rpn, W_score, b_score,
           W_bbox, b_bbox):
    f32 = jnp.float32
    patches = im_data[0].reshape(3, FEAT, STRIDE, FEAT, STRIDE)
    patches = patches.transpose(1, 3, 0, 2, 4).reshape(P, 3 * STRIDE * STRIDE)
    w1 = W_head.reshape(512, 3 * STRIDE * STRIDE).T
    b1 = jnp.broadcast_to(b_head[None, :], (8, 512))

    feat = _head_conv(patches, w1, b1)

    xpad = jnp.pad(feat.reshape(FEAT, FEAT, 512), ((1, 1), (1, 1), (0, 0)))
    xcol = jnp.concatenate(
        [xpad[dy:dy + FEAT, dx:dx + FEAT, :].reshape(P, 512)
         for dy in range(3) for dx in range(3)], axis=1)
    w2 = W_rpn.transpose(2, 3, 1, 0).reshape(9 * 512, 512)
    b2 = jnp.broadcast_to(b_rpn[None, :], (8, 512))
    rpn = _rpn_conv(xcol, w2, b2)

    ws = W_score.reshape(18, 512)
    wb = W_bbox.reshape(36, 512)
    perm = np.arange(36).reshape(9, 4).T.reshape(36)
    wb = wb[perm]
    wc = jnp.concatenate([ws, wb], axis=0).T
    wc = jnp.pad(wc, ((0, 0), (0, 10)))
    bc = jnp.concatenate([b_score, b_bbox[perm]])
    bc = jnp.broadcast_to(jnp.pad(bc, (0, 10))[None, :], (8, 64))

    anch = jnp.asarray(_ANCH)
    ax1 = anch[:, :, 0]
    ay1 = anch[:, :, 1]
    ax2 = anch[:, :, 2]
    ay2 = anch[:, :, 3]
    hw = jnp.zeros((8, 128), f32)
    hw = hw.at[:, 0].set(im_info[0, 0]).at[:, 1].set(im_info[0, 1])

    sc, px1, py1, px2, py2 = _heads(rpn, wc, bc, ax1, ay1, ax2, ay2, hw)

    sel = _nms(sc.reshape(ROWS, 128), px1.reshape(ROWS, 128),
               py1.reshape(ROWS, 128), px2.reshape(ROWS, 128),
               py2.reshape(ROWS, 128))

    rois = jnp.concatenate([jnp.zeros((POST_NMS_TOPN, 1), f32),
                            sel[:, 0:4]], axis=1)
    scores_keep = sel[:, 4:5]
    feature = feat.T.reshape(1, 512, FEAT, FEAT)
    return rois, scores_keep, feature
